# Initial kernel scaffold; baseline (speedup 1.0000x reference)
import jax
import jax.numpy as jnp
from jax.experimental import pallas as pl


def kernel(raw_features, wl_role_ids, init_pos_ids, hop_dis_ids, W, b, wl_table, pos_table, hop_table, gamma, beta):
    raise NotImplementedError("write your pallas kernel here")


# speedup vs baseline: 4.0510x; 4.0510x over previous
import functools

import jax
import jax.numpy as jnp
from jax import lax
from jax.experimental import pallas as pl
from jax.experimental.pallas import tpu as pltpu
from jax.experimental.pallas import tpu_sc as plsc

X_SIZE = 32
HIDDEN = 128
EPS = 1e-12

NC = 2
NS = 16
NW = NC * NS
NB = 128
BT = 2048


def _gather_sum_body(tok_per_w, wl_ids, pos_ids, hop_ids, wl_t, pos_t, hop_t,
                     out, iw, ip, ih, bw, bp, bh, s0, s1, s2):
    wid = lax.axis_index("s") * NC + lax.axis_index("c")
    base = wid * tok_per_w
    pltpu.sync_copy(wl_ids.at[pl.ds(base, tok_per_w)], iw)
    pltpu.sync_copy(pos_ids.at[pl.ds(base, tok_per_w)], ip)
    pltpu.sync_copy(hop_ids.at[pl.ds(base, tok_per_w)], ih)
    num_chunks = tok_per_w // NB

    def chunk(g, carry):
        off = base + g * NB
        cw = pltpu.async_copy(wl_t.at[iw.at[pl.ds(g * NB, NB)]], bw, s0)
        cp = pltpu.async_copy(pos_t.at[ip.at[pl.ds(g * NB, NB)]], bp, s1)
        ch = pltpu.async_copy(hop_t.at[ih.at[pl.ds(g * NB, NB)]], bh, s2)
        cw.wait()
        cp.wait()
        ch.wait()

        def add_row(t, c2):
            for c in range(HIDDEN // 16):
                sl = pl.ds(c * 16, 16)
                bw[t, sl] = bw[t, sl] + bp[t, sl] + bh[t, sl]
            return c2

        lax.fori_loop(0, NB, add_row, 0)
        pltpu.sync_copy(bw, out.at[pl.ds(off, NB)])
        return carry

    lax.fori_loop(0, num_chunks, chunk, 0)


def _proj_ln_body(raw_ref, gsum_ref, w_ref, b_ref, g_ref, be_ref, out_ref):
    proj = jnp.dot(raw_ref[...], w_ref[...], preferred_element_type=jnp.float32)
    e = proj + b_ref[...] + gsum_ref[...]
    mean = jnp.mean(e, axis=1, keepdims=True)
    cent = e - mean
    var = jnp.mean(cent * cent, axis=1, keepdims=True)
    normed = cent * lax.rsqrt(var + EPS)
    out_ref[...] = normed * g_ref[...] + be_ref[...]


def kernel(raw_features, wl_role_ids, init_pos_ids, hop_dis_ids, W, b,
           wl_table, pos_table, hop_table, gamma, beta):
    Bb, Ll, X = raw_features.shape
    N = Bb * Ll
    tok_per_w = N // NW

    raw2 = raw_features.reshape(N, X)
    wl_ids = wl_role_ids.reshape(N).astype(jnp.int32)
    pos_ids = init_pos_ids.reshape(N).astype(jnp.int32)
    hop_ids = hop_dis_ids.reshape(N).astype(jnp.int32)

    gather_fn = pl.kernel(
        functools.partial(_gather_sum_body, tok_per_w),
        out_type=jax.ShapeDtypeStruct((N, HIDDEN), jnp.float32),
        mesh=plsc.VectorSubcoreMesh(core_axis_name="c", subcore_axis_name="s"),
        scratch_types=[
            pltpu.VMEM((tok_per_w,), jnp.int32),
            pltpu.VMEM((tok_per_w,), jnp.int32),
            pltpu.VMEM((tok_per_w,), jnp.int32),
            pltpu.VMEM((NB, HIDDEN), jnp.float32),
            pltpu.VMEM((NB, HIDDEN), jnp.float32),
            pltpu.VMEM((NB, HIDDEN), jnp.float32),
            pltpu.SemaphoreType.DMA,
            pltpu.SemaphoreType.DMA,
            pltpu.SemaphoreType.DMA,
        ],
    )
    gsum = gather_fn(wl_ids, pos_ids, hop_ids, wl_table, pos_table, hop_table)

    out2 = pl.pallas_call(
        _proj_ln_body,
        grid=(N // BT,),
        in_specs=[
            pl.BlockSpec((BT, X), lambda i: (i, 0)),
            pl.BlockSpec((BT, HIDDEN), lambda i: (i, 0)),
            pl.BlockSpec((X_SIZE, HIDDEN), lambda i: (0, 0)),
            pl.BlockSpec((1, HIDDEN), lambda i: (0, 0)),
            pl.BlockSpec((1, HIDDEN), lambda i: (0, 0)),
            pl.BlockSpec((1, HIDDEN), lambda i: (0, 0)),
        ],
        out_specs=pl.BlockSpec((BT, HIDDEN), lambda i: (i, 0)),
        out_shape=jax.ShapeDtypeStruct((N, HIDDEN), jnp.float32),
    )(raw2, gsum, W, b.reshape(1, HIDDEN), gamma.reshape(1, HIDDEN),
      beta.reshape(1, HIDDEN))

    return out2.reshape(Bb, Ll, HIDDEN)


# speedup vs baseline: 4.3447x; 1.0725x over previous
import functools

import jax
import jax.numpy as jnp
from jax import lax
from jax.experimental import pallas as pl
from jax.experimental.pallas import tpu as pltpu
from jax.experimental.pallas import tpu_sc as plsc

X_SIZE = 32
HIDDEN = 128
EPS = 1e-12

NC = 2
NS = 16
NW = NC * NS
NB = 80
BT = 2048


def _gather_sum_body(tok_per_w, wl_ids, pos_ids, hop_ids, wl_t, pos_t, hop_t,
                     out, iw, ip, ih, bw0, bp0, bh0, bw1, bp1, bh1,
                     gs0, gs1, ss0, ss1):
    sid = lax.axis_index("s")
    wid = sid * NC + lax.axis_index("c")
    base = wid * tok_per_w
    num_chunks = tok_per_w // NB

    pltpu.sync_copy(wl_ids.at[pl.ds(base, tok_per_w)], iw)
    pltpu.sync_copy(pos_ids.at[pl.ds(base, tok_per_w)], ip)
    pltpu.sync_copy(hop_ids.at[pl.ds(base, tok_per_w)], ih)

    def issue(g, bw, bp, bh, sem):
        pltpu.async_copy(wl_t.at[iw.at[pl.ds(g * NB, NB)]], bw, sem)
        pltpu.async_copy(pos_t.at[ip.at[pl.ds(g * NB, NB)]], bp, sem)
        pltpu.async_copy(hop_t.at[ih.at[pl.ds(g * NB, NB)]], bh, sem)

    def drain_gathers(g, bw, bp, bh, sem):
        pltpu.make_async_copy(wl_t.at[iw.at[pl.ds(g * NB, NB)]], bw, sem).wait()
        pltpu.make_async_copy(pos_t.at[ip.at[pl.ds(g * NB, NB)]], bp, sem).wait()
        pltpu.make_async_copy(hop_t.at[ih.at[pl.ds(g * NB, NB)]], bh, sem).wait()

    def add_and_store(g, bw, bp, bh, sem):
        def row(t, c2):
            for cc in range(HIDDEN // 16):
                sl = pl.ds(cc * 16, 16)
                plsc.addupdate(bw.at[t, sl], bp[t, sl] + bh[t, sl])
            return c2

        lax.fori_loop(0, NB, row, 0)
        pltpu.async_copy(bw, out.at[pl.ds(base + g * NB, NB)], sem)

    def drain_scatter(bw, sem):
        pltpu.make_async_copy(bw, out.at[pl.ds(0, NB)], sem).wait()

    issue(0, bw0, bp0, bh0, gs0)

    def pair(k, carry):
        g0 = 2 * k
        g1 = g0 + 1

        @pl.when(k > 0)
        def _():
            drain_scatter(bw1, ss1)

        issue(g1, bw1, bp1, bh1, gs1)
        drain_gathers(g0, bw0, bp0, bh0, gs0)
        add_and_store(g0, bw0, bp0, bh0, ss0)

        @pl.when(k < num_chunks // 2 - 1)
        def _():
            drain_scatter(bw0, ss0)
            issue(g0 + 2, bw0, bp0, bh0, gs0)

        drain_gathers(g1, bw1, bp1, bh1, gs1)
        add_and_store(g1, bw1, bp1, bh1, ss1)
        return carry

    lax.fori_loop(0, num_chunks // 2, pair, 0)
    drain_scatter(bw0, ss0)
    drain_scatter(bw1, ss1)


def _proj_ln_body(raw_ref, gsum_ref, w_ref, b_ref, g_ref, be_ref, out_ref):
    proj = jnp.dot(raw_ref[...], w_ref[...], preferred_element_type=jnp.float32)
    e = proj + b_ref[...] + gsum_ref[...]
    mean = jnp.mean(e, axis=1, keepdims=True)
    cent = e - mean
    var = jnp.mean(cent * cent, axis=1, keepdims=True)
    normed = cent * lax.rsqrt(var + EPS)
    out_ref[...] = normed * g_ref[...] + be_ref[...]


def kernel(raw_features, wl_role_ids, init_pos_ids, hop_dis_ids, W, b,
           wl_table, pos_table, hop_table, gamma, beta):
    Bb, Ll, X = raw_features.shape
    N = Bb * Ll
    tok_per_w = N // NW

    raw2 = raw_features.reshape(N, X)
    wl_ids = wl_role_ids.reshape(N).astype(jnp.int32)
    pos_ids = init_pos_ids.reshape(N).astype(jnp.int32)
    hop_ids = hop_dis_ids.reshape(N).astype(jnp.int32)

    gather_fn = pl.kernel(
        functools.partial(_gather_sum_body, tok_per_w),
        out_type=jax.ShapeDtypeStruct((N, HIDDEN), jnp.float32),
        mesh=plsc.VectorSubcoreMesh(core_axis_name="c", subcore_axis_name="s"),
        scratch_types=[
            pltpu.VMEM((tok_per_w,), jnp.int32),
            pltpu.VMEM((tok_per_w,), jnp.int32),
            pltpu.VMEM((tok_per_w,), jnp.int32),
            pltpu.VMEM((NB, HIDDEN), jnp.float32),
            pltpu.VMEM((NB, HIDDEN), jnp.float32),
            pltpu.VMEM((NB, HIDDEN), jnp.float32),
            pltpu.VMEM((NB, HIDDEN), jnp.float32),
            pltpu.VMEM((NB, HIDDEN), jnp.float32),
            pltpu.VMEM((NB, HIDDEN), jnp.float32),
            pltpu.SemaphoreType.DMA,
            pltpu.SemaphoreType.DMA,
            pltpu.SemaphoreType.DMA,
            pltpu.SemaphoreType.DMA,
        ],
    )
    gsum = gather_fn(wl_ids, pos_ids, hop_ids, wl_table, pos_table, hop_table)

    out2 = pl.pallas_call(
        _proj_ln_body,
        grid=(N // BT,),
        in_specs=[
            pl.BlockSpec((BT, X), lambda i: (i, 0)),
            pl.BlockSpec((BT, HIDDEN), lambda i: (i, 0)),
            pl.BlockSpec((X_SIZE, HIDDEN), lambda i: (0, 0)),
            pl.BlockSpec((1, HIDDEN), lambda i: (0, 0)),
            pl.BlockSpec((1, HIDDEN), lambda i: (0, 0)),
            pl.BlockSpec((1, HIDDEN), lambda i: (0, 0)),
        ],
        out_specs=pl.BlockSpec((BT, HIDDEN), lambda i: (i, 0)),
        out_shape=jax.ShapeDtypeStruct((N, HIDDEN), jnp.float32),
    )(raw2, gsum, W, b.reshape(1, HIDDEN), gamma.reshape(1, HIDDEN),
      beta.reshape(1, HIDDEN))

    return out2.reshape(Bb, Ll, HIDDEN)
